# baseline (device time: 38202 ns/iter reference)
import jax
import jax.numpy as jnp
from jax import lax
from jax.experimental import pallas as pl
from jax.experimental.pallas import tpu as pltpu

N_DEV = 32
N_WAVE = 2


def kernel(x):
    m, n = x.shape
    mc = m // N_DEV
    mw = mc // N_WAVE

    def body(x_ref, out_ref, xb_ref, gather_ref, ag_ref,
             p1_send, p1_recv, p2_send, p2_recv, bar2_sem, bar3_sem):
        my = lax.axis_index("i")
        grp = (my // 4) * 4
        off = my % 4
        blk = (my // 16) * 16

        group_targets = [grp + (off + j) % 4 for j in range(1, 4)]
        cross_targets = [
            ((my // 4 + g) % 8) * 4 + (off + j) % 4
            for g in range(1, 8)
            for j in range(4)
        ]
        all_targets = group_targets + cross_targets

        def p1_desc(t, w):
            return pltpu.make_async_remote_copy(
                src_ref=xb_ref.at[pl.ds(t * mc + w * mw, mw), :],
                dst_ref=gather_ref.at[my, pl.ds(w * mw, mw), :],
                send_sem=p1_send.at[t, w],
                recv_sem=p1_recv.at[my, w],
                device_id=(t,),
                device_id_type=pl.DeviceIdType.MESH,
            )

        def p1_recv_desc(s, w):
            return pltpu.make_async_remote_copy(
                src_ref=xb_ref.at[pl.ds(s * mc + w * mw, mw), :],
                dst_ref=gather_ref.at[s, pl.ds(w * mw, mw), :],
                send_sem=p1_send.at[s, w],
                recv_sem=p1_recv.at[s, w],
                device_id=(s,),
                device_id_type=pl.DeviceIdType.MESH,
            )

        def p2_desc(t, w):
            return pltpu.make_async_remote_copy(
                src_ref=ag_ref.at[pl.ds(my * mc + w * mw, mw), :],
                dst_ref=ag_ref.at[pl.ds(my * mc + w * mw, mw), :],
                send_sem=p2_send.at[t, w],
                recv_sem=p2_recv.at[my, w],
                device_id=(t,),
                device_id_type=pl.DeviceIdType.MESH,
            )

        def p2_recv_desc(s, w):
            return pltpu.make_async_remote_copy(
                src_ref=ag_ref.at[pl.ds(s * mc + w * mw, mw), :],
                dst_ref=ag_ref.at[pl.ds(s * mc + w * mw, mw), :],
                send_sem=p2_send.at[s, w],
                recv_sem=p2_recv.at[s, w],
                device_id=(s,),
                device_id_type=pl.DeviceIdType.MESH,
            )

        bar_sem = pltpu.get_barrier_semaphore()
        for t in group_targets:
            pl.semaphore_signal(
                bar_sem, inc=1,
                device_id=(t,), device_id_type=pl.DeviceIdType.MESH,
            )

        xb_ref[...] = x_ref[...].astype(jnp.bfloat16)

        pl.semaphore_wait(bar_sem, 3)

        for w in range(N_WAVE):
            for t in group_targets:
                p1_desc(t, w).start()

        for j in range(1, 4):
            pl.semaphore_signal(
                bar2_sem, inc=1,
                device_id=(blk + (my + 4 * j) % 16,),
                device_id_type=pl.DeviceIdType.MESH,
            )
        pl.semaphore_wait(bar2_sem, 3)

        pl.semaphore_signal(
            bar3_sem, inc=1,
            device_id=((my + 16) % N_DEV,),
            device_id_type=pl.DeviceIdType.MESH,
        )
        pl.semaphore_wait(bar3_sem, 1)

        for w in range(N_WAVE):
            for t in cross_targets:
                p1_desc(t, w).start()

        red_waves = []
        for w in range(N_WAVE):
            for s in all_targets:
                p1_recv_desc(s, w).wait_recv()
                p1_desc(s, w).wait_send()

            gb = gather_ref[:, pl.ds(w * mw, mw), :].astype(jnp.float32)
            slot = lax.broadcasted_iota(jnp.int32, gb.shape, 0)
            own = x_ref[pl.ds(my * mc + w * mw, mw), :]
            red = jnp.where(slot == my, 0.0, gb).sum(axis=0) + own
            red_waves.append(red)
            ag_ref[pl.ds(my * mc + w * mw, mw), :] = red.astype(jnp.bfloat16)

            for t in all_targets:
                p2_desc(t, w).start()

        for w in range(N_WAVE):
            for s in all_targets:
                p2_recv_desc(s, w).wait_recv()
                p2_desc(s, w).wait_send()

        out_ref[...] = ag_ref[...].astype(jnp.float32)
        for w in range(N_WAVE):
            out_ref[pl.ds(my * mc + w * mw, mw), :] = red_waves[w]

    return pl.pallas_call(
        body,
        out_shape=jax.ShapeDtypeStruct((m, n), x.dtype),
        in_specs=[pl.BlockSpec(memory_space=pltpu.VMEM)],
        out_specs=pl.BlockSpec(memory_space=pltpu.VMEM),
        scratch_shapes=[
            pltpu.VMEM((m, n), jnp.bfloat16),
            pltpu.VMEM((N_DEV, mc, n), jnp.bfloat16),
            pltpu.VMEM((m, n), jnp.bfloat16),
            pltpu.SemaphoreType.DMA((N_DEV, N_WAVE)),
            pltpu.SemaphoreType.DMA((N_DEV, N_WAVE)),
            pltpu.SemaphoreType.DMA((N_DEV, N_WAVE)),
            pltpu.SemaphoreType.DMA((N_DEV, N_WAVE)),
            pltpu.SemaphoreType.REGULAR,
            pltpu.SemaphoreType.REGULAR,
        ],
        compiler_params=pltpu.CompilerParams(collective_id=0),
    )(x)
